# baseline (speedup 1.0000x reference)
import numpy as np

import jax
import jax.numpy as jnp
from jax import lax
from jax.experimental import pallas as pl
from jax.experimental.pallas import tpu as pltpu
from jax.experimental.pallas import tpu_sc as plsc

N = 10000
D_IN = 128
D_H = 128
D_A = 64
D_CAT = D_H + D_A
HALF = D_CAT // 2
HALF_I32 = HALF // 2
K = 64

NC, NS = 2, 16
NW = NC * NS
CH = 128
N_PAD = 10240
E_PAD = 327680
CPT_A = E_PAD // (NW * CH)
CH3 = 80
CPT_C = E_PAD // (NS * CH3)
NPH = 8
HPH = CPT_C // NPH
RPT = N_PAD // NS

_PERM = np.concatenate(
    [np.stack([np.arange(16) + 32 * g, np.arange(16) + 32 * g + 16], axis=1).reshape(-1)
     for g in range(HALF // 32)]
)


def _deg_body(dst_hbm, out_hbm, idx_v, ones_v, zero_v, acc_sh, sems):
    c = lax.axis_index("c")
    s = lax.axis_index("s")
    wid = c * NS + s

    def fill_ones(i, _):
        ones_v[pl.ds(i * 16, 16)] = jnp.ones((16,), jnp.float32)
        return 0

    lax.fori_loop(0, CH // 16, fill_ones, 0)

    def fill_zero(i, _):
        zero_v[pl.ds(i * 16, 16)] = jnp.zeros((16,), jnp.float32)
        return 0

    lax.fori_loop(0, RPT // 16, fill_zero, 0)

    pltpu.sync_copy(zero_v, acc_sh.at[pl.ds(s * RPT, RPT)])
    pltpu.sync_copy(dst_hbm.at[pl.ds(wid * CPT_A, CPT_A)], idx_v)
    plsc.subcore_barrier()

    def ring(t, _):
        j0 = t * 4
        for b in range(4):
            j = j0 + b

            @pl.when(j >= 4)
            def _wait():
                pltpu.make_async_copy(ones_v, acc_sh.at[idx_v.at[j]], sems[b]).wait()

            pltpu.async_copy(ones_v, acc_sh.at[idx_v.at[j]], sems[b], add=True)
        return 0

    lax.fori_loop(0, CPT_A // 4, ring, 0)
    for b in range(4):
        j = CPT_A - 4 + b
        pltpu.make_async_copy(ones_v, acc_sh.at[idx_v.at[j]], sems[b]).wait()

    plsc.subcore_barrier()
    pltpu.sync_copy(
        acc_sh.at[pl.ds(s * RPT, RPT)],
        out_hbm.at[c, pl.ds(s * RPT, RPT)],
    )


_deg_call = pl.kernel(
    _deg_body,
    out_type=jax.ShapeDtypeStruct((NC, N_PAD), jnp.float32),
    mesh=plsc.VectorSubcoreMesh(core_axis_name="c", subcore_axis_name="s"),
    compiler_params=pltpu.CompilerParams(use_tc_tiling_on_sc=False),
    scratch_types=[
        pltpu.VMEM((CPT_A, CH), jnp.int32),
        pltpu.VMEM((CH,), jnp.float32),
        pltpu.VMEM((RPT,), jnp.float32),
        pltpu.VMEM_SHARED((N_PAD,), jnp.float32),
        [pltpu.SemaphoreType.DMA] * 4,
    ],
)


def _mm_body(deg_ref, x_ref, w0p_ref, w1p_ref, y0b_ref, y1b_ref):
    dinv = lax.rsqrt(deg_ref[...])
    xs = x_ref[...] * dinv
    y0b_ref[...] = jnp.dot(
        xs, w0p_ref[...], preferred_element_type=jnp.float32
    ).astype(jnp.bfloat16)
    y1b_ref[...] = jnp.dot(
        xs, w1p_ref[...], preferred_element_type=jnp.float32
    ).astype(jnp.bfloat16)


_MM_BLK = 512


def _mm_call(deg_col, x, w0p, w1p):
    return pl.pallas_call(
        _mm_body,
        grid=(N_PAD // _MM_BLK,),
        in_specs=[
            pl.BlockSpec((_MM_BLK, 1), lambda i: (i, 0)),
            pl.BlockSpec((_MM_BLK, D_IN), lambda i: (i, 0)),
            pl.BlockSpec((D_IN, HALF), lambda i: (0, 0)),
            pl.BlockSpec((D_IN, HALF), lambda i: (0, 0)),
        ],
        out_specs=[
            pl.BlockSpec((_MM_BLK, HALF), lambda i: (i, 0)),
            pl.BlockSpec((_MM_BLK, HALF), lambda i: (i, 0)),
        ],
        out_shape=[
            jax.ShapeDtypeStruct((N_PAD, HALF), jnp.bfloat16),
            jax.ShapeDtypeStruct((N_PAD, HALF), jnp.bfloat16),
        ],
    )(deg_col, x, w0p, w1p)


def _convert(bbuf, fbuf, rows):
    def row(r, _):
        for g in range(HALF // 32):
            v = bbuf[r, pl.ds(16 * g, 16)]
            lo = lax.bitcast_convert_type(lax.shift_left(v, 16), jnp.float32)
            hi = lax.bitcast_convert_type(
                lax.bitwise_and(v, jnp.int32(-65536)), jnp.float32
            )
            fbuf[r, pl.ds(32 * g, 16)] = lo
            fbuf[r, pl.ds(32 * g + 16, 16)] = hi
        return 0

    lax.fori_loop(0, rows, row, 0, unroll=4)


def _scatter_body(y0i_hbm, y1i_hbm, src_hbm, dst_hbm, out_hbm,
                  src_v, dst_v, bbuf0, bbuf1, fbuf0, fbuf1, ysp, acc_sh,
                  gsem0, gsem1, ssem0, ssem1):
    c = lax.axis_index("c")
    s = lax.axis_index("s")

    bbufs = (bbuf0, bbuf1)
    fbufs = (fbuf0, fbuf1)
    gsems = (gsem0, gsem1)
    ssems = (ssem0, ssem1)

    def run(yi_ref):
        pltpu.sync_copy(yi_ref.at[pl.ds(s * RPT, RPT)], ysp.at[pl.ds(s * RPT, RPT)])
        for q in range(RPT // CH3):
            base = s * RPT + q * CH3
            pltpu.sync_copy(ysp.at[pl.ds(base, CH3)], bbuf0)
            _convert(bbuf0, fbuf0, CH3)
            pltpu.sync_copy(fbuf0, acc_sh.at[pl.ds(base, CH3)])
        plsc.subcore_barrier()

        for p in range(NPH):
            if p > 0:
                for b in range(2):
                    pltpu.make_async_copy(fbufs[b], acc_sh.at[dst_v.at[0]], ssems[b]).wait()
            pltpu.sync_copy(src_hbm.at[pl.ds(s * CPT_C + p * HPH, HPH)], src_v)
            pltpu.sync_copy(dst_hbm.at[pl.ds(s * CPT_C + p * HPH, HPH)], dst_v)
            pltpu.async_copy(ysp.at[src_v.at[0]], bbufs[0], gsems[0])
            pltpu.async_copy(ysp.at[src_v.at[1]], bbufs[1], gsems[1])

            def ring(t2, _):
                for b in range(2):
                    t = t2 * 2 + b

                    @pl.when(t >= 2)
                    def _fwait():
                        pltpu.make_async_copy(
                            fbufs[b], acc_sh.at[dst_v.at[t]], ssems[b]
                        ).wait()

                    pltpu.make_async_copy(
                        ysp.at[src_v.at[t]], bbufs[b], gsems[b]
                    ).wait()
                    _convert(bbufs[b], fbufs[b], CH3)

                    @pl.when(t + 2 < HPH)
                    def _next():
                        pltpu.async_copy(ysp.at[src_v.at[t + 2]], bbufs[b], gsems[b])

                    pltpu.async_copy(fbufs[b], acc_sh.at[dst_v.at[t]], ssems[b], add=True)
                return 0

            lax.fori_loop(0, HPH // 2, ring, 0)

        for b in range(2):
            pltpu.make_async_copy(fbufs[b], acc_sh.at[dst_v.at[0]], ssems[b]).wait()

    @pl.when(c == 0)
    def _run0():
        run(y0i_hbm)

    @pl.when(c == 1)
    def _run1():
        run(y1i_hbm)

    plsc.subcore_barrier()
    pltpu.sync_copy(
        acc_sh.at[pl.ds(s * RPT, RPT)],
        out_hbm.at[c, pl.ds(s * RPT, RPT)],
    )


_scatter_call = pl.kernel(
    _scatter_body,
    out_type=jax.ShapeDtypeStruct((NC, N_PAD, HALF), jnp.float32),
    mesh=plsc.VectorSubcoreMesh(core_axis_name="c", subcore_axis_name="s"),
    compiler_params=pltpu.CompilerParams(use_tc_tiling_on_sc=False),
    scratch_types=[
        pltpu.VMEM((HPH, CH3), jnp.int32),
        pltpu.VMEM((HPH, CH3), jnp.int32),
        pltpu.VMEM((CH3, HALF_I32), jnp.int32),
        pltpu.VMEM((CH3, HALF_I32), jnp.int32),
        pltpu.VMEM((CH3, HALF), jnp.float32),
        pltpu.VMEM((CH3, HALF), jnp.float32),
        pltpu.VMEM_SHARED((N_PAD, HALF_I32), jnp.int32),
        pltpu.VMEM_SHARED((N_PAD, HALF), jnp.float32),
        pltpu.SemaphoreType.DMA,
        pltpu.SemaphoreType.DMA,
        pltpu.SemaphoreType.DMA,
        pltpu.SemaphoreType.DMA,
    ],
)


_F_BLK = 1000


def _final_body(acc_ref, deg_ref, b0_ref, b1_ref, wlp_ref, bl_ref,
                out_ref, pa_acc, pb_acc):
    i = pl.program_id(0)
    dinv = lax.rsqrt(deg_ref[...])
    r0 = jnp.maximum(acc_ref[0] * dinv + b0_ref[...], 0.0)
    r1 = jnp.maximum(acc_ref[1] * dinv + b1_ref[...], 0.0)
    logits = jnp.dot(r1, wlp_ref[...], preferred_element_type=jnp.float32) + bl_ref[...]
    m = jnp.max(logits, axis=1, keepdims=True)
    e = jnp.exp(logits - m)
    sm = e / jnp.sum(e, axis=1, keepdims=True)
    pa = lax.dot_general(sm, r0, (((0,), (0,)), ((), ())),
                         preferred_element_type=jnp.float32)
    pb = lax.dot_general(sm, r1, (((0,), (0,)), ((), ())),
                         preferred_element_type=jnp.float32)

    @pl.when(i == 0)
    def _init():
        pa_acc[...] = jnp.zeros_like(pa_acc)
        pb_acc[...] = jnp.zeros_like(pb_acc)

    pa_acc[...] += pa
    pb_acc[...] += pb

    @pl.when(i == pl.num_programs(0) - 1)
    def _out():
        out_ref[...] = jnp.concatenate([pa_acc[...], pb_acc[:, :D_H - HALF]], axis=1)


def _final_call(acc, deg_col, b0, b1, wlp, bl):
    return pl.pallas_call(
        _final_body,
        grid=(N // _F_BLK,),
        in_specs=[
            pl.BlockSpec((NC, _F_BLK, HALF), lambda i: (0, i, 0)),
            pl.BlockSpec((_F_BLK, 1), lambda i: (i, 0)),
            pl.BlockSpec((1, HALF), lambda i: (0, 0)),
            pl.BlockSpec((1, HALF), lambda i: (0, 0)),
            pl.BlockSpec((HALF, K), lambda i: (0, 0)),
            pl.BlockSpec((1, K), lambda i: (0, 0)),
        ],
        out_specs=pl.BlockSpec((K, D_H), lambda i: (0, 0)),
        out_shape=jax.ShapeDtypeStruct((K, D_H), jnp.float32),
        scratch_shapes=[
            pltpu.VMEM((K, HALF), jnp.float32),
            pltpu.VMEM((K, HALF), jnp.float32),
        ],
    )(acc, deg_col, b0, b1, wlp, bl)


def kernel(x, edge_index, W_embed, b_embed, W_assign, b_assign, W_lin, b_lin):
    E = edge_index.shape[1]
    w_cat = jnp.concatenate([W_embed, W_assign], axis=1)
    w0 = w_cat[:, :HALF]
    w1 = w_cat[:, HALF:]
    perm = jnp.asarray(_PERM)
    w0p = w0[:, perm]
    w1p = w1[:, perm]
    pad = E_PAD - E
    src_flat = jnp.concatenate([edge_index[0], jnp.zeros((pad,), jnp.int32)])
    dst_flat = jnp.concatenate([edge_index[1], jnp.full((pad,), N, jnp.int32)])
    dst = dst_flat.reshape(E_PAD // CH, CH)
    src_c = src_flat.reshape(E_PAD // CH3, CH3)
    dst_c = dst_flat.reshape(E_PAD // CH3, CH3)

    b_cat = jnp.concatenate([b_embed, b_assign])
    b0 = b_cat[:HALF].reshape(1, HALF)
    b1 = b_cat[HALF:].reshape(1, HALF)
    wlp = jnp.concatenate(
        [jnp.zeros((D_CAT - HALF - D_A, K), jnp.float32), W_lin], axis=0
    )

    deg_parts = _deg_call(dst)
    deg_col = (deg_parts[0] + deg_parts[1] + 1.0).reshape(N_PAD, 1)
    y0b, y1b = _mm_call(deg_col, x, w0p, w1p)
    y0i = lax.bitcast_convert_type(y0b.reshape(N_PAD, HALF_I32, 2), jnp.int32)
    y1i = lax.bitcast_convert_type(y1b.reshape(N_PAD, HALF_I32, 2), jnp.int32)
    acc = _scatter_call(y0i, y1i, src_c, dst_c)
    x_pool = _final_call(acc, deg_col, b0, b1, wlp, b_lin.reshape(1, K))
    return x_pool


# speedup vs baseline: 1.0202x; 1.0202x over previous
import numpy as np

import jax
import jax.numpy as jnp
from jax import lax
from jax.experimental import pallas as pl
from jax.experimental.pallas import tpu as pltpu
from jax.experimental.pallas import tpu_sc as plsc

N = 10000
D_IN = 128
D_H = 128
D_A = 64
D_CAT = D_H + D_A
HALF = D_CAT // 2
HALF_I32 = HALF // 2
K = 64

NC, NS = 2, 16
NW = NC * NS
CH = 128
N_PAD = 10240
E_PAD = 327680
CPT_A = E_PAD // (NW * CH)
CH3 = 64
CPT_C = E_PAD // (NS * CH3)
NPH = 4
HPH = CPT_C // NPH
RPT = N_PAD // NS

_PERM = np.concatenate(
    [np.stack([np.arange(16) + 32 * g, np.arange(16) + 32 * g + 16], axis=1).reshape(-1)
     for g in range(HALF // 32)]
)


def _deg_body(dst_hbm, out_hbm, idx_v, ones_v, zero_v, acc_sh, sems):
    c = lax.axis_index("c")
    s = lax.axis_index("s")
    wid = c * NS + s

    def fill_ones(i, _):
        ones_v[pl.ds(i * 16, 16)] = jnp.ones((16,), jnp.float32)
        return 0

    lax.fori_loop(0, CH // 16, fill_ones, 0)

    def fill_zero(i, _):
        zero_v[pl.ds(i * 16, 16)] = jnp.zeros((16,), jnp.float32)
        return 0

    lax.fori_loop(0, RPT // 16, fill_zero, 0)

    pltpu.sync_copy(zero_v, acc_sh.at[pl.ds(s * RPT, RPT)])
    pltpu.sync_copy(dst_hbm.at[pl.ds(wid * CPT_A, CPT_A)], idx_v)
    plsc.subcore_barrier()

    def ring(t, _):
        j0 = t * 4
        for b in range(4):
            j = j0 + b

            @pl.when(j >= 4)
            def _wait():
                pltpu.make_async_copy(ones_v, acc_sh.at[idx_v.at[j]], sems[b]).wait()

            pltpu.async_copy(ones_v, acc_sh.at[idx_v.at[j]], sems[b], add=True)
        return 0

    lax.fori_loop(0, CPT_A // 4, ring, 0)
    for b in range(4):
        j = CPT_A - 4 + b
        pltpu.make_async_copy(ones_v, acc_sh.at[idx_v.at[j]], sems[b]).wait()

    plsc.subcore_barrier()
    pltpu.sync_copy(
        acc_sh.at[pl.ds(s * RPT, RPT)],
        out_hbm.at[c, pl.ds(s * RPT, RPT)],
    )


_deg_call = pl.kernel(
    _deg_body,
    out_type=jax.ShapeDtypeStruct((NC, N_PAD), jnp.float32),
    mesh=plsc.VectorSubcoreMesh(core_axis_name="c", subcore_axis_name="s"),
    compiler_params=pltpu.CompilerParams(use_tc_tiling_on_sc=False),
    scratch_types=[
        pltpu.VMEM((CPT_A, CH), jnp.int32),
        pltpu.VMEM((CH,), jnp.float32),
        pltpu.VMEM((RPT,), jnp.float32),
        pltpu.VMEM_SHARED((N_PAD,), jnp.float32),
        [pltpu.SemaphoreType.DMA] * 4,
    ],
)


def _mm_body(deg_ref, x_ref, w0p_ref, w1p_ref, y0b_ref, y1b_ref):
    dinv = lax.rsqrt(deg_ref[...])
    xs = x_ref[...] * dinv
    y0b_ref[...] = jnp.dot(
        xs, w0p_ref[...], preferred_element_type=jnp.float32
    ).astype(jnp.bfloat16)
    y1b_ref[...] = jnp.dot(
        xs, w1p_ref[...], preferred_element_type=jnp.float32
    ).astype(jnp.bfloat16)


_MM_BLK = 512


def _mm_call(deg_col, x, w0p, w1p):
    return pl.pallas_call(
        _mm_body,
        grid=(N_PAD // _MM_BLK,),
        in_specs=[
            pl.BlockSpec((_MM_BLK, 1), lambda i: (i, 0)),
            pl.BlockSpec((_MM_BLK, D_IN), lambda i: (i, 0)),
            pl.BlockSpec((D_IN, HALF), lambda i: (0, 0)),
            pl.BlockSpec((D_IN, HALF), lambda i: (0, 0)),
        ],
        out_specs=[
            pl.BlockSpec((_MM_BLK, HALF), lambda i: (i, 0)),
            pl.BlockSpec((_MM_BLK, HALF), lambda i: (i, 0)),
        ],
        out_shape=[
            jax.ShapeDtypeStruct((N_PAD, HALF), jnp.bfloat16),
            jax.ShapeDtypeStruct((N_PAD, HALF), jnp.bfloat16),
        ],
    )(deg_col, x, w0p, w1p)


def _convert(bbuf, fbuf, rows):
    def row(r, _):
        for g in range(HALF // 32):
            v = bbuf[r, pl.ds(16 * g, 16)]
            lo = lax.bitcast_convert_type(lax.shift_left(v, 16), jnp.float32)
            hi = lax.bitcast_convert_type(
                lax.bitwise_and(v, jnp.int32(-65536)), jnp.float32
            )
            fbuf[r, pl.ds(32 * g, 16)] = lo
            fbuf[r, pl.ds(32 * g + 16, 16)] = hi
        return 0

    lax.fori_loop(0, rows, row, 0, unroll=4)


def _scatter_body(y0i_hbm, y1i_hbm, src_hbm, dst_hbm, out_hbm,
                  src_v, dst_v, bbuf0, bbuf1, fbuf0, fbuf1, ysp, acc_sh,
                  gsem0, gsem1, ssem0, ssem1):
    c = lax.axis_index("c")
    s = lax.axis_index("s")

    bbufs = (bbuf0, bbuf1)
    fbufs = (fbuf0, fbuf1)
    gsems = (gsem0, gsem1)
    ssems = (ssem0, ssem1)

    def run(yi_ref):
        pltpu.sync_copy(yi_ref.at[pl.ds(s * RPT, RPT)], ysp.at[pl.ds(s * RPT, RPT)])
        for q in range(RPT // CH3):
            base = s * RPT + q * CH3
            pltpu.sync_copy(ysp.at[pl.ds(base, CH3)], bbuf0)
            _convert(bbuf0, fbuf0, CH3)
            pltpu.sync_copy(fbuf0, acc_sh.at[pl.ds(base, CH3)])
        plsc.subcore_barrier()

        for p in range(NPH):
            if p > 0:
                for b in range(2):
                    pltpu.make_async_copy(fbufs[b], acc_sh.at[dst_v.at[0]], ssems[b]).wait()
            pltpu.sync_copy(src_hbm.at[pl.ds(s * CPT_C + p * HPH, HPH)], src_v)
            pltpu.sync_copy(dst_hbm.at[pl.ds(s * CPT_C + p * HPH, HPH)], dst_v)
            pltpu.async_copy(ysp.at[src_v.at[0]], bbufs[0], gsems[0])
            pltpu.async_copy(ysp.at[src_v.at[1]], bbufs[1], gsems[1])

            def ring(t2, _):
                for b in range(2):
                    t = t2 * 2 + b

                    @pl.when(t >= 2)
                    def _fwait():
                        pltpu.make_async_copy(
                            fbufs[b], acc_sh.at[dst_v.at[t]], ssems[b]
                        ).wait()

                    pltpu.make_async_copy(
                        ysp.at[src_v.at[t]], bbufs[b], gsems[b]
                    ).wait()
                    _convert(bbufs[b], fbufs[b], CH3)

                    @pl.when(t + 2 < HPH)
                    def _next():
                        pltpu.async_copy(ysp.at[src_v.at[t + 2]], bbufs[b], gsems[b])

                    pltpu.async_copy(fbufs[b], acc_sh.at[dst_v.at[t]], ssems[b], add=True)
                return 0

            lax.fori_loop(0, HPH // 2, ring, 0)

        for b in range(2):
            pltpu.make_async_copy(fbufs[b], acc_sh.at[dst_v.at[0]], ssems[b]).wait()

    @pl.when(c == 0)
    def _run0():
        run(y0i_hbm)

    @pl.when(c == 1)
    def _run1():
        run(y1i_hbm)

    plsc.subcore_barrier()
    pltpu.sync_copy(
        acc_sh.at[pl.ds(s * RPT, RPT)],
        out_hbm.at[c, pl.ds(s * RPT, RPT)],
    )


_scatter_call = pl.kernel(
    _scatter_body,
    out_type=jax.ShapeDtypeStruct((NC, N_PAD, HALF), jnp.float32),
    mesh=plsc.VectorSubcoreMesh(core_axis_name="c", subcore_axis_name="s"),
    compiler_params=pltpu.CompilerParams(use_tc_tiling_on_sc=False),
    scratch_types=[
        pltpu.VMEM((HPH, CH3), jnp.int32),
        pltpu.VMEM((HPH, CH3), jnp.int32),
        pltpu.VMEM((CH3, HALF_I32), jnp.int32),
        pltpu.VMEM((CH3, HALF_I32), jnp.int32),
        pltpu.VMEM((CH3, HALF), jnp.float32),
        pltpu.VMEM((CH3, HALF), jnp.float32),
        pltpu.VMEM_SHARED((N_PAD, HALF_I32), jnp.int32),
        pltpu.VMEM_SHARED((N_PAD, HALF), jnp.float32),
        pltpu.SemaphoreType.DMA,
        pltpu.SemaphoreType.DMA,
        pltpu.SemaphoreType.DMA,
        pltpu.SemaphoreType.DMA,
    ],
)


_F_BLK = 1000


def _final_body(acc_ref, deg_ref, b0_ref, b1_ref, wlp_ref, bl_ref,
                out_ref, pa_acc, pb_acc):
    i = pl.program_id(0)
    dinv = lax.rsqrt(deg_ref[...])
    r0 = jnp.maximum(acc_ref[0] * dinv + b0_ref[...], 0.0)
    r1 = jnp.maximum(acc_ref[1] * dinv + b1_ref[...], 0.0)
    logits = jnp.dot(r1, wlp_ref[...], preferred_element_type=jnp.float32) + bl_ref[...]
    m = jnp.max(logits, axis=1, keepdims=True)
    e = jnp.exp(logits - m)
    sm = e / jnp.sum(e, axis=1, keepdims=True)
    pa = lax.dot_general(sm, r0, (((0,), (0,)), ((), ())),
                         preferred_element_type=jnp.float32)
    pb = lax.dot_general(sm, r1, (((0,), (0,)), ((), ())),
                         preferred_element_type=jnp.float32)

    @pl.when(i == 0)
    def _init():
        pa_acc[...] = jnp.zeros_like(pa_acc)
        pb_acc[...] = jnp.zeros_like(pb_acc)

    pa_acc[...] += pa
    pb_acc[...] += pb

    @pl.when(i == pl.num_programs(0) - 1)
    def _out():
        out_ref[...] = jnp.concatenate([pa_acc[...], pb_acc[:, :D_H - HALF]], axis=1)


def _final_call(acc, deg_col, b0, b1, wlp, bl):
    return pl.pallas_call(
        _final_body,
        grid=(N // _F_BLK,),
        in_specs=[
            pl.BlockSpec((NC, _F_BLK, HALF), lambda i: (0, i, 0)),
            pl.BlockSpec((_F_BLK, 1), lambda i: (i, 0)),
            pl.BlockSpec((1, HALF), lambda i: (0, 0)),
            pl.BlockSpec((1, HALF), lambda i: (0, 0)),
            pl.BlockSpec((HALF, K), lambda i: (0, 0)),
            pl.BlockSpec((1, K), lambda i: (0, 0)),
        ],
        out_specs=pl.BlockSpec((K, D_H), lambda i: (0, 0)),
        out_shape=jax.ShapeDtypeStruct((K, D_H), jnp.float32),
        scratch_shapes=[
            pltpu.VMEM((K, HALF), jnp.float32),
            pltpu.VMEM((K, HALF), jnp.float32),
        ],
    )(acc, deg_col, b0, b1, wlp, bl)


def kernel(x, edge_index, W_embed, b_embed, W_assign, b_assign, W_lin, b_lin):
    E = edge_index.shape[1]
    w_cat = jnp.concatenate([W_embed, W_assign], axis=1)
    w0 = w_cat[:, :HALF]
    w1 = w_cat[:, HALF:]
    perm = jnp.asarray(_PERM)
    w0p = w0[:, perm]
    w1p = w1[:, perm]
    pad = E_PAD - E
    src_flat = jnp.concatenate([edge_index[0], jnp.zeros((pad,), jnp.int32)])
    dst_flat = jnp.concatenate([edge_index[1], jnp.full((pad,), N, jnp.int32)])
    dst = dst_flat.reshape(E_PAD // CH, CH)
    src_c = src_flat.reshape(E_PAD // CH3, CH3)
    dst_c = dst_flat.reshape(E_PAD // CH3, CH3)

    b_cat = jnp.concatenate([b_embed, b_assign])
    b0 = b_cat[:HALF].reshape(1, HALF)
    b1 = b_cat[HALF:].reshape(1, HALF)
    wlp = jnp.concatenate(
        [jnp.zeros((D_CAT - HALF - D_A, K), jnp.float32), W_lin], axis=0
    )

    deg_parts = _deg_call(dst)
    deg_col = (deg_parts[0] + deg_parts[1] + 1.0).reshape(N_PAD, 1)
    y0b, y1b = _mm_call(deg_col, x, w0p, w1p)
    y0i = lax.bitcast_convert_type(y0b.reshape(N_PAD, HALF_I32, 2), jnp.int32)
    y1i = lax.bitcast_convert_type(y1b.reshape(N_PAD, HALF_I32, 2), jnp.int32)
    acc = _scatter_call(y0i, y1i, src_c, dst_c)
    x_pool = _final_call(acc, deg_col, b0, b1, wlp, b_lin.reshape(1, K))
    return x_pool
